# baseline (speedup 1.0000x reference)
import functools

import jax
import jax.numpy as jnp
from jax import lax
from jax.experimental import pallas as pl
from jax.experimental.pallas import tpu as pltpu
from jax.experimental.pallas import tpu_sc as plsc

_N = 10000
_M = 32
_ORIG = 128
_A = 32
_NB = 4
_K = 3
_NCONV = 3
_H = 128
_N0 = 100
_P = 100
_D = 2 * _A + _NB
_E = _N * _M

def _b16(x):
    return x.astype(jnp.bfloat16).astype(jnp.float32)


def _dot(a, b):
    return lax.dot(_b16(a), _b16(b), preferred_element_type=jnp.float32)


_NW = 32
_BPW = _E // _NW
_CH = 2000
_NCH = _BPW // _CH


def _sc_gather(table, idx_flat):
    mesh = plsc.VectorSubcoreMesh(core_axis_name="c", subcore_axis_name="s")

    @functools.partial(
        pl.kernel,
        mesh=mesh,
        out_type=jax.ShapeDtypeStruct((_E, _A), jnp.float32),
        scratch_types=[
            pltpu.VMEM((_CH,), jnp.int32),
            pltpu.VMEM((_CH, _A), jnp.float32),
            pltpu.SemaphoreType.DMA,
        ],
        compiler_params=pltpu.CompilerParams(use_tc_tiling_on_sc=False),
    )
    def gk(table_hbm, idx_hbm, out_hbm, idx_v, rows_v, sem):
        wid = lax.axis_index("s") * 2 + lax.axis_index("c")
        base = wid * _BPW

        def body(j, carry):
            off = base + j * _CH
            pltpu.sync_copy(idx_hbm.at[pl.ds(off, _CH)], idx_v)
            pltpu.async_copy(table_hbm.at[idx_v], rows_v, sem).wait()
            pltpu.sync_copy(rows_v, out_hbm.at[pl.ds(off, _CH)])
            return carry

        lax.fori_loop(0, _NCH, body, 0)

    return gk(table, idx_flat)


def _embed(atom_fea, embT, emb_b):
    tn = 2000

    def body(x_ref, w_ref, b_ref, o_ref):
        o_ref[...] = _dot(x_ref[...], w_ref[...]) + b_ref[...]

    return pl.pallas_call(
        body,
        grid=(_N // tn,),
        in_specs=[
            pl.BlockSpec((tn, _ORIG), lambda i: (i, 0)),
            pl.BlockSpec((_ORIG, _A), lambda i: (0, 0)),
            pl.BlockSpec((1, _A), lambda i: (0, 0)),
        ],
        out_specs=pl.BlockSpec((tn, _A), lambda i: (i, 0)),
        out_shape=jax.ShapeDtypeStruct((_N, _A), jnp.float32),
    )(atom_fea, embT, emb_b)


_TA = 400


def _gram_pass(gath, nbr_e, atom):
    r = _TA * _M

    def body(g_ref, e_ref, a_ref, gram_ref, cs_ref):
        at = a_ref[...]
        selfb = jnp.broadcast_to(at[:, None, :], (_TA, _M, _A)).reshape(r, _A)
        ne = _b16(jnp.concatenate([selfb, g_ref[...], e_ref[...]], axis=1))
        gacc = lax.dot_general(ne, ne, (((0,), (0,)), ((), ())),
                               preferred_element_type=jnp.float32)
        csacc = jnp.sum(ne, axis=0, keepdims=True)

        @pl.when(pl.program_id(0) == 0)
        def _():
            gram_ref[...] = gacc
            cs_ref[...] = csacc

        @pl.when(pl.program_id(0) != 0)
        def _():
            gram_ref[...] += gacc
            cs_ref[...] += csacc

    return pl.pallas_call(
        body,
        grid=(_N // _TA,),
        in_specs=[
            pl.BlockSpec((r, _A), lambda i: (i, 0)),
            pl.BlockSpec((r, _NB), lambda i: (i, 0)),
            pl.BlockSpec((_TA, _A), lambda i: (i, 0)),
        ],
        out_specs=[
            pl.BlockSpec((_D, _D), lambda i: (0, 0)),
            pl.BlockSpec((1, _D), lambda i: (0, 0)),
        ],
        out_shape=[
            jax.ShapeDtypeStruct((_D, _D), jnp.float32),
            jax.ShapeDtypeStruct((1, _D), jnp.float32),
        ],
        compiler_params=pltpu.CompilerParams(
            dimension_semantics=("arbitrary",)),
    )(gath, nbr_e, atom)


_TB = 200


_KA = _K * _A
_NNW = _K * _NB
_WB = 224
_CO = 128


def _conv_pass(gath, nbr_e, atom, wbig, scale_b, shift_b, g24, nb24):
    r = _TB * _M

    def body(g_ref, e_ref, a_ref, w_ref, sc_ref, sh_ref, g24_ref, nb_ref,
             ns_ref, nn_ref, t1_ref, t2_ref):
        em = e_ref[...]
        at = a_ref[...]
        selfb = jnp.broadcast_to(at[:, None, :], (_TB, _M, _A)).reshape(r, _A)
        ne16 = _b16(jnp.concatenate([selfb, g_ref[...], em], axis=1))
        tg = lax.dot(ne16, w_ref[...], preferred_element_type=jnp.float32)
        tg = tg * sc_ref[...] + sh_ref[...]
        p = jnp.exp(tg[:, :_KA])
        core = jnp.maximum(tg[:, _CO:_CO + _KA], 0.0)
        z = jnp.sum(p.reshape(_TB, _M, _KA), axis=1)
        pcs = jnp.sum((p * core).reshape(_TB, _M, _KA), axis=1)
        ns = pcs / z
        ns_ref[...] = ns
        nnb = tg[:, _KA:_KA + _NNW] + jnp.concatenate([em, em, em], axis=1)
        ga = lax.dot(_b16(nnb), g24_ref[...],
                     preferred_element_type=jnp.float32) + nb_ref[...]
        f3 = ga[:, 12:16]
        f4 = ga[:, 16:20]
        f5 = ga[:, 20:24]
        fm = jnp.maximum(jnp.maximum(f3, f4), f5)
        ex3 = jnp.exp(f3 - fm)
        ex4 = jnp.exp(f4 - fm)
        ex5 = jnp.exp(f5 - fm)
        nn_ref[...] = ((ga[:, 0:4] * ex3 + ga[:, 4:8] * ex4
                        + ga[:, 8:12] * ex5) / (ex3 + ex4 + ex5))
        a1 = jnp.sum(ns, axis=0, keepdims=True)
        a2 = jnp.sum(ns * ns, axis=0, keepdims=True)

        @pl.when(pl.program_id(0) == 0)
        def _():
            t1_ref[...] = a1
            t2_ref[...] = a2

        @pl.when(pl.program_id(0) != 0)
        def _():
            t1_ref[...] += a1
            t2_ref[...] += a2

    return pl.pallas_call(
        body,
        grid=(_N // _TB,),
        in_specs=[
            pl.BlockSpec((r, _A), lambda i: (i, 0)),
            pl.BlockSpec((r, _NB), lambda i: (i, 0)),
            pl.BlockSpec((_TB, _A), lambda i: (i, 0)),
            pl.BlockSpec((_D, _WB), lambda i: (0, 0)),
            pl.BlockSpec((1, _WB), lambda i: (0, 0)),
            pl.BlockSpec((1, _WB), lambda i: (0, 0)),
            pl.BlockSpec((_NNW, 24), lambda i: (0, 0)),
            pl.BlockSpec((1, 24), lambda i: (0, 0)),
        ],
        out_specs=[
            pl.BlockSpec((_TB, _KA), lambda i: (i, 0)),
            pl.BlockSpec((r, _NB), lambda i: (i, 0)),
            pl.BlockSpec((1, _KA), lambda i: (0, 0)),
            pl.BlockSpec((1, _KA), lambda i: (0, 0)),
        ],
        out_shape=[
            jax.ShapeDtypeStruct((_N, _KA), jnp.float32),
            jax.ShapeDtypeStruct((_E, _NB), jnp.float32),
            jax.ShapeDtypeStruct((1, _KA), jnp.float32),
            jax.ShapeDtypeStruct((1, _KA), jnp.float32),
        ],
        compiler_params=pltpu.CompilerParams(
            dimension_semantics=("arbitrary",)),
    )(gath, nbr_e, atom, wbig, scale_b, shift_b, g24, nb24)


_TU = 2000


def _update_pass(atom, nsum, scale2, shift2, aw, ab):
    def body(a_ref, ns_ref, sc_ref, sh_ref, aw_ref, ab_ref, o_ref):
        at = a_ref[...]
        outk = []
        for i in range(_K):
            v = (ns_ref[:, i * _A:(i + 1) * _A] * sc_ref[i:i + 1, :]
                 + sh_ref[i:i + 1, :])
            outk.append(at + v)
        ok = [_b16(v) for v in outk]
        g = []
        for j in range(2 * _K):
            g.append(ok[0] * aw_ref[j, 0] + ok[1] * aw_ref[j, 1]
                     + ok[2] * aw_ref[j, 2] + ab_ref[0, j])
        fm = jnp.maximum(jnp.maximum(g[3], g[4]), g[5])
        e3 = jnp.exp(g[3] - fm)
        e4 = jnp.exp(g[4] - fm)
        e5 = jnp.exp(g[5] - fm)
        z = e3 + e4 + e5
        o_ref[...] = (g[0] * e3 + g[1] * e4 + g[2] * e5) / z

    return pl.pallas_call(
        body,
        grid=(_N // _TU,),
        in_specs=[
            pl.BlockSpec((_TU, _A), lambda i: (i, 0)),
            pl.BlockSpec((_TU, _K * _A), lambda i: (i, 0)),
            pl.BlockSpec((_K, _A), lambda i: (0, 0)),
            pl.BlockSpec((_K, _A), lambda i: (0, 0)),
            pl.BlockSpec(memory_space=pltpu.SMEM),
            pl.BlockSpec(memory_space=pltpu.SMEM),
        ],
        out_specs=pl.BlockSpec((_TU, _A), lambda i: (i, 0)),
        out_shape=jax.ShapeDtypeStruct((_N, _A), jnp.float32),
    )(atom, nsum, scale2, shift2, aw, ab)


def _pool_head(x, fc1t, fc1_b, outt, out_b):
    def body(x_ref, w1_ref, b1_ref, w2_ref, b2_ref, o_ref):
        crys = jnp.sum(x_ref[...].reshape(_N0, _P, _A), axis=1) / float(_P)
        crys = jnp.maximum(crys, 0.0)
        h = jnp.maximum(_dot(crys, w1_ref[...]) + b1_ref[...], 0.0)
        o_ref[...] = _dot(h, w2_ref[...]) + b2_ref[...]

    return pl.pallas_call(
        body,
        grid=(1,),
        in_specs=[
            pl.BlockSpec((_N, _A), lambda i: (0, 0)),
            pl.BlockSpec((_A, _H), lambda i: (0, 0)),
            pl.BlockSpec((1, _H), lambda i: (0, 0)),
            pl.BlockSpec((_H, 1), lambda i: (0, 0)),
            pl.BlockSpec((1, 1), lambda i: (0, 0)),
        ],
        out_specs=pl.BlockSpec((_N0, 1), lambda i: (0, 0)),
        out_shape=jax.ShapeDtypeStruct((_N0, 1), jnp.float32),
    )(x, fc1t, fc1_b, outt, out_b)


def kernel(atom_fea, nbr_fea, nbr_fea_idx, crystal_atom_idx, emb_W, emb_b,
           fc_full_W, fc_full_b, bn1_g, bn1_b, bn2_g, bn2_b,
           atom_fc_W, atom_fc_b, nbr_fc_W, nbr_fc_b, fc1_W, fc1_b,
           out_W, out_b):
    idx_flat = nbr_fea_idx.reshape(-1).astype(jnp.int32)
    x = _embed(atom_fea, emb_W.T, emb_b.reshape(1, _A))
    nbr_e = nbr_fea.reshape(_E, _NB)
    re = float(_E)
    hi = lax.Precision.HIGHEST
    for c in range(_NCONV):
        wt16 = _b16(jnp.swapaxes(fc_full_W[c], 1, 2))
        gath = _sc_gather(x, idx_flat)
        gram, cs = _gram_pass(gath, nbr_e, x)
        mu = jnp.concatenate(
            [lax.dot(cs, wt16[i], precision=hi) for i in range(_K)],
            axis=0) / re
        gw = [lax.dot(gram, wt16[i], precision=hi) for i in range(_K)]
        m2 = jnp.stack([jnp.sum(gw[i] * wt16[i], axis=0)
                        for i in range(_K)]) / re
        var = m2 - mu * mu
        scale1 = bn1_g[c] / jnp.sqrt(var + 1e-5)
        shift1 = bn1_b[c] - mu * scale1
        pad = jnp.zeros((_D, _CO - _KA - _NNW), jnp.float32)
        wbig = jnp.concatenate(
            [jnp.concatenate([wt16[i][:, :_A] for i in range(_K)], axis=1),
             jnp.concatenate([wt16[i][:, 2 * _A:] for i in range(_K)], axis=1),
             pad,
             jnp.concatenate([wt16[i][:, _A:2 * _A] for i in range(_K)],
                             axis=1)],
            axis=1)
        padr = jnp.zeros((1, _CO - _KA - _NNW), jnp.float32)

        def _packrow(m):
            return jnp.concatenate(
                [m[:, :_A].reshape(1, _KA), m[:, 2 * _A:].reshape(1, _NNW),
                 padr, m[:, _A:2 * _A].reshape(1, _KA)], axis=1)

        g24 = jnp.kron(_b16(nbr_fc_W[c]).T, jnp.eye(_NB, dtype=jnp.float32))
        nb24 = jnp.repeat(nbr_fc_b[c], _NB).reshape(1, 2 * _K * _NB)
        ns, nn, t1, t2 = _conv_pass(
            gath, nbr_e, x, wbig, _packrow(scale1), _packrow(shift1),
            g24, nb24)
        mu2 = t1.reshape(_K, _A) / float(_N)
        var2 = t2.reshape(_K, _A) / float(_N) - mu2 * mu2
        scale2 = bn2_g[c] / jnp.sqrt(var2 + 1e-5)
        shift2 = bn2_b[c] - mu2 * scale2
        x = _update_pass(x, ns, scale2, shift2,
                         _b16(atom_fc_W[c]), atom_fc_b[c].reshape(1, 2 * _K))
        nbr_e = nn
    return _pool_head(x, fc1_W.T, fc1_b.reshape(1, _H),
                      out_W.T, out_b.reshape(1, 1))
